# Initial kernel scaffold; baseline (speedup 1.0000x reference)
import jax
import jax.numpy as jnp
from jax.experimental import pallas as pl


def kernel(dag_x, dag_edge_index, res_x, res_edge_index, params):
    raise NotImplementedError("write your pallas kernel here")


# speedup vs baseline: 8.4377x; 8.4377x over previous
import jax
import jax.numpy as jnp
from jax import lax
from jax.experimental import pallas as pl
from jax.experimental.pallas import tpu as pltpu
from jax.experimental.pallas import tpu_sc as plsc

F32 = jnp.float32
NC, NS = 2, 16
W = 1000

ND, ED = 50000, 800000
NR, ER = 10000, 160000
NDP = 50048
NRP = 10112
H = 64

_SC_PARAMS = pltpu.CompilerParams(use_tc_tiling_on_sc=False)


def _chunks(rows_t):
    out, off = [], 0
    while off < rows_t:
        sz = min(1000, rows_t - off)
        out.append((off, sz))
        off += sz
    return out


def _dotg(a, b):
    return lax.dot_general(a, b, (((1,), (1,)), ((), ())),
                           preferred_element_type=F32)


def _make_sc_seg16_dirsplit(n_pad, e):
    e_t = e // NS
    nwin = e_t // W
    rows_t = n_pad // NS
    chunks = _chunks(rows_t)
    mesh = plsc.VectorSubcoreMesh(core_axis_name="c", subcore_axis_name="s")

    def body(src_h, dst_h, xp_h, zeros_h, outf_h, outb_h,
             idx_s, idx_d, rows, stage, sem, acc):
        c = lax.axis_index("c")
        s = lax.axis_index("s")
        r0 = s * rows_t
        e0 = s * e_t

        pltpu.sync_copy(zeros_h, stage)
        for off, sz in chunks:
            pltpu.sync_copy(stage.at[pl.ds(0, sz)],
                            acc.at[pl.ds(r0 + off, sz)])
        plsc.subcore_barrier()

        def accum(gref, sref):
            def win(k, carry):
                base = e0 + k * W
                pltpu.sync_copy(src_h.at[pl.ds(base, W)], idx_s)
                pltpu.sync_copy(dst_h.at[pl.ds(base, W)], idx_d)
                pltpu.async_copy(xp_h.at[gref], rows, sem).wait()
                pltpu.sync_copy(rows, acc.at[sref], add=True)
                return carry
            lax.fori_loop(0, nwin, win, 0)

        @pl.when(c == 0)
        def _():
            accum(idx_s, idx_d)

        @pl.when(c == 1)
        def _():
            accum(idx_d, idx_s)

        plsc.subcore_barrier()

        def flush(out_h):
            for off, sz in chunks:
                pltpu.sync_copy(acc.at[pl.ds(r0 + off, sz)],
                                stage.at[pl.ds(0, sz)])
                pltpu.sync_copy(stage.at[pl.ds(0, sz)],
                                out_h.at[pl.ds(r0 + off, sz)])

        @pl.when(c == 0)
        def _():
            flush(outf_h)

        @pl.when(c == 1)
        def _():
            flush(outb_h)

    out_t = jax.ShapeDtypeStruct((n_pad, 16), F32)
    scratch = [
        pltpu.VMEM((W,), jnp.int32),
        pltpu.VMEM((W,), jnp.int32),
        pltpu.VMEM((W, 16), F32),
        pltpu.VMEM((1000, 16), F32),
        pltpu.SemaphoreType.DMA,
        pltpu.VMEM_SHARED((n_pad, 16), F32),
    ]
    return pl.kernel(
        body, out_type=(out_t, out_t), mesh=mesh, scratch_types=scratch,
        compiler_params=_SC_PARAMS)


def _make_sc_seg16_edgesplit(n_pad, e):
    e_w = e // (NC * NS)
    nwin = e_w // W
    rows_t = n_pad // NS
    chunks = _chunks(rows_t)
    mesh = plsc.VectorSubcoreMesh(core_axis_name="c", subcore_axis_name="s")

    def body(src_h, dst_h, xp_h, zeros_h, out0_h, out1_h,
             idx_s, idx_d, rows, stage, sem, acc):
        c = lax.axis_index("c")
        s = lax.axis_index("s")
        w = s * NC + c
        r0 = s * rows_t
        e0 = w * e_w

        pltpu.sync_copy(zeros_h, stage)
        for off, sz in chunks:
            pltpu.sync_copy(stage.at[pl.ds(0, sz)],
                            acc.at[pl.ds(r0 + off, sz)])
        plsc.subcore_barrier()

        def win(k, carry):
            base = e0 + k * W
            pltpu.sync_copy(src_h.at[pl.ds(base, W)], idx_s)
            pltpu.sync_copy(dst_h.at[pl.ds(base, W)], idx_d)
            pltpu.async_copy(xp_h.at[idx_s], rows, sem).wait()
            pltpu.sync_copy(rows, acc.at[idx_d], add=True)
            return carry

        lax.fori_loop(0, nwin, win, 0)
        plsc.subcore_barrier()

        def flush(out_h):
            for off, sz in chunks:
                pltpu.sync_copy(acc.at[pl.ds(r0 + off, sz)],
                                stage.at[pl.ds(0, sz)])
                pltpu.sync_copy(stage.at[pl.ds(0, sz)],
                                out_h.at[pl.ds(r0 + off, sz)])

        @pl.when(c == 0)
        def _():
            flush(out0_h)

        @pl.when(c == 1)
        def _():
            flush(out1_h)

    out_t = jax.ShapeDtypeStruct((n_pad, 16), F32)
    scratch = [
        pltpu.VMEM((W,), jnp.int32),
        pltpu.VMEM((W,), jnp.int32),
        pltpu.VMEM((W, 16), F32),
        pltpu.VMEM((1000, 16), F32),
        pltpu.SemaphoreType.DMA,
        pltpu.VMEM_SHARED((n_pad, 16), F32),
    ]
    return pl.kernel(
        body, out_type=(out_t, out_t), mesh=mesh, scratch_types=scratch,
        compiler_params=_SC_PARAMS)


def _make_sc_seg64(n_pad, e, dual):
    ndir = 2 if dual else 1
    e_t = e // NS
    nwin = e_t // W
    rows_t = n_pad // NS
    chunks = _chunks(rows_t)
    mesh = plsc.VectorSubcoreMesh(core_axis_name="c", subcore_axis_name="s")

    def body(src_h, dst_h, h0, h1, h2, h3, zeros_h, *rest):
        nout = 4 * ndir
        outs = rest[:nout]
        idx_s, idx_d, rows, zbuf, fbuf, sem, acc = rest[nout:]
        tables = (h0, h1, h2, h3)
        c = lax.axis_index("c")
        s = lax.axis_index("s")
        r0 = s * rows_t
        e0 = s * e_t

        pltpu.sync_copy(zeros_h, zbuf)

        def zero_acc():
            for off, sz in chunks:
                pltpu.sync_copy(zbuf.at[pl.ds(0, sz)],
                                acc.at[pl.ds(r0 + off, sz)])

        def accum(tab_h, gref, sref):
            def win(k, carry):
                base = e0 + k * W
                pltpu.sync_copy(src_h.at[pl.ds(base, W)], idx_s)
                pltpu.sync_copy(dst_h.at[pl.ds(base, W)], idx_d)
                pltpu.async_copy(tab_h.at[gref], rows, sem).wait()
                pltpu.sync_copy(rows, acc.at[sref], add=True)
                return carry
            lax.fori_loop(0, nwin, win, 0)

        def flush(out_h):
            for off, sz in chunks:
                pltpu.sync_copy(acc.at[pl.ds(r0 + off, sz)],
                                fbuf.at[pl.ds(0, sz)])
                pltpu.sync_copy(fbuf.at[pl.ds(0, sz)],
                                out_h.at[pl.ds(r0 + off, sz)])

        for d in range(ndir):
            for pq in range(2):
                zero_acc()
                plsc.subcore_barrier()

                @pl.when(c == 0)
                def _(d=d, pq=pq):
                    gref, sref = (idx_s, idx_d) if d == 0 else (idx_d, idx_s)
                    accum(tables[2 * pq], gref, sref)

                @pl.when(c == 1)
                def _(d=d, pq=pq):
                    gref, sref = (idx_s, idx_d) if d == 0 else (idx_d, idx_s)
                    accum(tables[2 * pq + 1], gref, sref)

                plsc.subcore_barrier()

                @pl.when(c == 0)
                def _(d=d, pq=pq):
                    flush(outs[d * 4 + 2 * pq])

                @pl.when(c == 1)
                def _(d=d, pq=pq):
                    flush(outs[d * 4 + 2 * pq + 1])

    out_t = jax.ShapeDtypeStruct((n_pad, 16), F32)
    scratch = [
        pltpu.VMEM((W,), jnp.int32),
        pltpu.VMEM((W,), jnp.int32),
        pltpu.VMEM((W, 16), F32),
        pltpu.VMEM((1000, 16), F32),
        pltpu.VMEM((1000, 16), F32),
        pltpu.SemaphoreType.DMA,
        pltpu.VMEM_SHARED((n_pad, 16), F32),
    ]
    return pl.kernel(
        body, out_type=tuple([out_t] * (4 * ndir)),
        mesh=mesh, scratch_types=scratch,
        compiler_params=_SC_PARAMS)


def _make_tc1(n, r, ndir, cc, partial):
    nb = n // r
    nslot = 2 if partial else ndir

    def body(*refs):
        ps = refs[:nslot]
        xp = refs[nslot]
        wls = refs[nslot + 1: nslot + 1 + ndir]
        wx = refs[nslot + 1 + ndir]
        hpre, stats, cnts, acc = refs[nslot + 2 + ndir:]
        i = pl.program_id(0)

        sf = (ps[0][...] + ps[1][...]) if partial else ps[0][...]
        cf = jnp.maximum(sf[:, cc:cc + 1], 1.0)
        h = _dotg(sf / cf, wls[0][...])
        if ndir == 2:
            sb = ps[1][...]
            cb = jnp.maximum(sb[:, cc:cc + 1], 1.0)
            h = h + _dotg(sb / cb, wls[1][...])
        else:
            cb = jnp.zeros_like(cf)
        h = h + _dotg(xp[...], wx[...])
        hpre[...] = h
        cnts[...] = jnp.concatenate([cf, cb, jnp.zeros((r, 6), F32)], 1)

        bs = jnp.sum(h, axis=0, keepdims=True)
        bq = jnp.sum(h * h, axis=0, keepdims=True)

        @pl.when(i == 0)
        def _():
            acc[0:1, :] = bs
            acc[1:2, :] = bq

        @pl.when(i > 0)
        def _():
            acc[0:1, :] = acc[0:1, :] + bs
            acc[1:2, :] = acc[1:2, :] + bq

        mu = acc[0:1, :] / float(n)
        var = jnp.maximum(acc[1:2, :] / float(n) - mu * mu, 0.0)
        stats[...] = jnp.concatenate([mu, var], 0)

    in_specs = (
        [pl.BlockSpec((r, 16), lambda i: (i, 0))] * (nslot + 1)
        + [pl.BlockSpec((H, 16), lambda i: (0, 0))] * (ndir + 1)
    )
    out_specs = [
        pl.BlockSpec((r, H), lambda i: (i, 0)),
        pl.BlockSpec((2, H), lambda i: (0, 0)),
        pl.BlockSpec((r, 8), lambda i: (i, 0)),
    ]
    out_shape = [
        jax.ShapeDtypeStruct((n, H), F32),
        jax.ShapeDtypeStruct((2, H), F32),
        jax.ShapeDtypeStruct((n, 8), F32),
    ]
    return pl.pallas_call(
        body, grid=(nb,), in_specs=in_specs, out_specs=out_specs,
        out_shape=out_shape,
        scratch_shapes=[pltpu.VMEM((8, H), F32)])


def _make_tc_bn(n, r):
    nb = n // r

    def body(hpre, stats, g, b, *outs):
        mu = stats[0:1, :]
        var = stats[1:2, :]
        y = g[...] * (hpre[...] - mu) * lax.rsqrt(var + 1e-5) + b[...]
        y = jnp.maximum(y, 0.0)
        for q in range(4):
            outs[q][...] = y[:, 16 * q:16 * (q + 1)]

    return pl.pallas_call(
        body, grid=(nb,),
        in_specs=[
            pl.BlockSpec((r, H), lambda i: (i, 0)),
            pl.BlockSpec((2, H), lambda i: (0, 0)),
            pl.BlockSpec((1, H), lambda i: (0, 0)),
            pl.BlockSpec((1, H), lambda i: (0, 0)),
        ],
        out_specs=[pl.BlockSpec((r, 16), lambda i: (i, 0))] * 4,
        out_shape=[jax.ShapeDtypeStruct((n, 16), F32)] * 4)


def _make_tc2(n, r, ndir):
    nb = n // r
    nseg = 4 * ndir

    def body(*refs):
        segs = refs[:nseg]
        hq = refs[nseg:nseg + 4]
        cnts = refs[nseg + 4]
        wls = refs[nseg + 5:nseg + 5 + ndir]
        wr = refs[nseg + 5 + ndir]
        bias = refs[nseg + 6 + ndir]
        h2, stats, acc = refs[nseg + 7 + ndir:]
        i = pl.program_id(0)

        cf = cnts[:, 0:1]
        af = jnp.concatenate([segs[q][...] for q in range(4)], 1) / cf
        h = _dotg(af, wls[0][...])
        if ndir == 2:
            cb = cnts[:, 1:2]
            ab = jnp.concatenate([segs[4 + q][...] for q in range(4)], 1) / cb
            h = h + _dotg(ab, wls[1][...])
        hprev = jnp.concatenate([hq[q][...] for q in range(4)], 1)
        h = h + _dotg(hprev, wr[...]) + bias[...]
        h2[...] = h

        bs = jnp.sum(h, axis=0, keepdims=True)
        bq_ = jnp.sum(h * h, axis=0, keepdims=True)

        @pl.when(i == 0)
        def _():
            acc[0:1, :] = bs
            acc[1:2, :] = bq_

        @pl.when(i > 0)
        def _():
            acc[0:1, :] = acc[0:1, :] + bs
            acc[1:2, :] = acc[1:2, :] + bq_

        mu = acc[0:1, :] / float(n)
        var = jnp.maximum(acc[1:2, :] / float(n) - mu * mu, 0.0)
        stats[...] = jnp.concatenate([mu, var], 0)

    in_specs = (
        [pl.BlockSpec((r, 16), lambda i: (i, 0))] * (nseg + 4)
        + [pl.BlockSpec((r, 8), lambda i: (i, 0))]
        + [pl.BlockSpec((H, H), lambda i: (0, 0))] * (ndir + 1)
        + [pl.BlockSpec((1, H), lambda i: (0, 0))]
    )
    return pl.pallas_call(
        body, grid=(nb,), in_specs=in_specs,
        out_specs=[
            pl.BlockSpec((r, H), lambda i: (i, 0)),
            pl.BlockSpec((2, H), lambda i: (0, 0)),
        ],
        out_shape=[
            jax.ShapeDtypeStruct((n, H), F32),
            jax.ShapeDtypeStruct((2, H), F32),
        ],
        scratch_shapes=[pltpu.VMEM((8, H), F32)])


def _make_tc_bnmax(n, r):
    nb = n // r

    def body(hpre, stats, g, b, emb, acc):
        i = pl.program_id(0)
        mu = stats[0:1, :]
        var = stats[1:2, :]
        y = g[...] * (hpre[...] - mu) * lax.rsqrt(var + 1e-5) + b[...]
        y = jnp.maximum(y, 0.0)
        m = jnp.max(y, axis=0, keepdims=True)

        @pl.when(i == 0)
        def _():
            acc[0:1, :] = m

        @pl.when(i > 0)
        def _():
            acc[0:1, :] = jnp.maximum(acc[0:1, :], m)

        emb[...] = acc[0:1, :]

    return pl.pallas_call(
        body, grid=(nb,),
        in_specs=[
            pl.BlockSpec((r, H), lambda i: (i, 0)),
            pl.BlockSpec((2, H), lambda i: (0, 0)),
            pl.BlockSpec((1, H), lambda i: (0, 0)),
            pl.BlockSpec((1, H), lambda i: (0, 0)),
        ],
        out_specs=pl.BlockSpec((1, H), lambda i: (0, 0)),
        out_shape=jax.ShapeDtypeStruct((1, H), F32),
        scratch_shapes=[pltpu.VMEM((8, H), F32)])


def _joint_body(de, re_, w, b, out):
    j = jnp.concatenate([de[...], re_[...]], 1)
    out[...] = jnp.maximum(_dotg(j, w[...]) + b[...], 0.0)


_joint_call = pl.pallas_call(
    _joint_body,
    out_shape=jax.ShapeDtypeStruct((1, 128), F32))


def _pad16(wcols, bias=None, bias_col=None):
    out = jnp.zeros((H, 16), F32).at[:, :wcols.shape[1]].set(wcols)
    if bias is not None:
        out = out.at[:, bias_col].set(bias)
    return out


def kernel(dag_x, dag_edge_index, res_x, res_edge_index, params):
    p = params
    dsrc, ddst = dag_edge_index[0], dag_edge_index[1]
    rsrc, rdst = res_edge_index[0], res_edge_index[1]

    xp_d = jnp.concatenate(
        [dag_x, jnp.ones((ND, 1), F32), jnp.zeros((ND, 10), F32)], 1)
    xp_r = jnp.concatenate(
        [res_x, jnp.ones((NR, 1), F32), jnp.zeros((NR, 13), F32)], 1)
    z16 = jnp.zeros((1000, 16), F32)

    sd_f, sd_b = _make_sc_seg16_dirsplit(NDP, ED)(dsrc, ddst, xp_d, z16)
    pr0, pr1 = _make_sc_seg16_edgesplit(NRP, ER)(rsrc, rdst, xp_r, z16)

    wlf1 = _pad16(p['d_f1_Wl'])
    wlb1 = _pad16(p['d_b1_Wl'])
    wx1 = _pad16(p['d_f1_Wr'] + p['d_b1_Wr'],
                 bias=p['d_f1_b'] + p['d_b1_b'], bias_col=5)
    hpre_d, stats_d, cnts_d = _make_tc1(ND, 2000, 2, 5, False)(
        sd_f, sd_b, xp_d, wlf1, wlb1, wx1)
    hq_d = _make_tc_bn(ND, 2000)(
        hpre_d, stats_d, p['d_bn1_g'].reshape(1, H),
        p['d_bn1_b'].reshape(1, H))

    wlr1 = _pad16(p['r_c1_Wl'])
    wxr1 = _pad16(p['r_c1_Wr'], bias=p['r_c1_b'], bias_col=2)
    hpre_r, stats_r, cnts_r = _make_tc1(NR, 2000, 1, 2, True)(
        pr0, pr1, xp_r, wlr1, wxr1)
    hq_r = _make_tc_bn(NR, 2000)(
        hpre_r, stats_r, p['r_bn1_g'].reshape(1, H),
        p['r_bn1_b'].reshape(1, H))

    s2_d = _make_sc_seg64(NDP, ED, True)(dsrc, ddst, *hq_d, z16)
    s2_r = _make_sc_seg64(NRP, ER, False)(rsrc, rdst, *hq_r, z16)

    h2_d, stats2_d = _make_tc2(ND, 2000, 2)(
        *s2_d, *hq_d, cnts_d,
        p['d_f2_Wl'], p['d_b2_Wl'], p['d_f2_Wr'] + p['d_b2_Wr'],
        (p['d_f2_b'] + p['d_b2_b']).reshape(1, H))
    demb = _make_tc_bnmax(ND, 2000)(
        h2_d, stats2_d, p['d_bn2_g'].reshape(1, H),
        p['d_bn2_b'].reshape(1, H))

    h2_r, stats2_r = _make_tc2(NR, 2000, 1)(
        *s2_r, *hq_r, cnts_r,
        p['r_c2_Wl'], p['r_c2_Wr'], p['r_c2_b'].reshape(1, H))
    remb = _make_tc_bnmax(NR, 2000)(
        h2_r, stats2_r, p['r_bn2_g'].reshape(1, H),
        p['r_bn2_b'].reshape(1, H))

    out = _joint_call(demb, remb, p['j_W'], p['j_b'].reshape(1, 128))
    return out.reshape(128)
